# Initial kernel scaffold; baseline (speedup 1.0000x reference)
import jax
import jax.numpy as jnp
from jax.experimental import pallas as pl


def kernel(x, pe_table):
    raise NotImplementedError("write your pallas kernel here")


# speedup vs baseline: 1.2764x; 1.2764x over previous
import jax
import jax.numpy as jnp
from jax.experimental import pallas as pl

_MAX_LEN = 8192
_PE_DIM = 128
_D_MODEL = 1024
_SEQ_BLK = 512


def _body(x_ref, pe_ref, o_ref):
    o_ref[:, :, :_D_MODEL] = x_ref[...]
    o_ref[:, :, _D_MODEL:] = pe_ref[...][None]


def kernel(x, pe_table):
    batch, max_len, d_model = x.shape
    pe_dim = pe_table.shape[1]
    grid = (batch, max_len // _SEQ_BLK)
    return pl.pallas_call(
        _body,
        grid=grid,
        in_specs=[
            pl.BlockSpec((1, _SEQ_BLK, d_model), lambda b, s: (b, s, 0)),
            pl.BlockSpec((_SEQ_BLK, pe_dim), lambda b, s: (s, 0)),
        ],
        out_specs=pl.BlockSpec((1, _SEQ_BLK, d_model + pe_dim),
                               lambda b, s: (b, s, 0)),
        out_shape=jax.ShapeDtypeStruct((batch, max_len, d_model + pe_dim),
                                       x.dtype),
    )(x, pe_table)
